# Initial kernel scaffold; baseline (speedup 1.0000x reference)
import jax
import jax.numpy as jnp
from jax.experimental import pallas as pl


def kernel(x, edge_index, batch_num_nodes, W0, b0, W1, b1, Wlin, blin):
    raise NotImplementedError("write your pallas kernel here")


# speedup vs baseline: 4.0416x; 4.0416x over previous
import functools

import jax
import jax.numpy as jnp
from jax import lax
from jax.experimental import pallas as pl
from jax.experimental.pallas import tpu as pltpu
from jax.experimental.pallas import tpu_sc as plsc

N = 10000
E = 320000
D = 128
H = 64
C = 40
G = 10

NC, NS = 2, 16
NW = NC * NS
CH = 128

NPAD = 10240
ROWS_PER_TILE = NPAD // NS

DEG_LEN = 2 * NPAD
DEG_PER_TILE = ((2 * E // NW + CH - 1) // CH) * CH
DEG_EPAD = DEG_PER_TILE * NW
DEG_STEPS = DEG_PER_TILE // CH
DEG_SLICE = DEG_LEN // NS
DEG_DUMMY = 10100

EPT = ((E // NW + CH - 1) // CH) * CH
EPAD = EPT * NW
MSG_STEPS = EPT // CH
MSG_DUMMY_DST = 10200

_MESH = plsc.VectorSubcoreMesh(core_axis_name="c", subcore_axis_name="s")
_SC_PARAMS = pltpu.CompilerParams(use_tc_tiling_on_sc=False)


@functools.partial(
    pl.kernel,
    out_type=jax.ShapeDtypeStruct((NC, DEG_LEN), jnp.float32),
    mesh=_MESH,
    scratch_types=[
        pltpu.VMEM((CH,), jnp.int32),
        pltpu.VMEM((CH,), jnp.float32),
        pltpu.VMEM((DEG_SLICE,), jnp.float32),
        pltpu.VMEM_SHARED((DEG_LEN,), jnp.float32),
    ],
    compiler_params=_SC_PARAMS,
)
def _deg_kernel(idx_hbm, out_hbm, idx_v, ones_v, zer_v, acc):
    cid = lax.axis_index("c")
    sid = lax.axis_index("s")
    wid = cid * NS + sid

    @pl.loop(0, DEG_SLICE // 16)
    def _zero(i):
        zer_v[pl.ds(i * 16, 16)] = jnp.zeros((16,), jnp.float32)

    @pl.loop(0, CH // 16)
    def _ones(i):
        ones_v[pl.ds(i * 16, 16)] = jnp.ones((16,), jnp.float32)

    pltpu.sync_copy(zer_v, acc.at[pl.ds(sid * DEG_SLICE, DEG_SLICE)])
    plsc.subcore_barrier()

    base = wid * DEG_PER_TILE

    @pl.loop(0, DEG_STEPS)
    def _step(j):
        pltpu.sync_copy(idx_hbm.at[pl.ds(base + j * CH, CH)], idx_v)
        pltpu.sync_copy(ones_v, acc.at[idx_v], add=True)

    plsc.subcore_barrier()
    pltpu.sync_copy(acc.at[pl.ds(sid * DEG_SLICE, DEG_SLICE)],
                    out_hbm.at[cid, pl.ds(sid * DEG_SLICE, DEG_SLICE)])


@functools.partial(
    pl.kernel,
    out_type=jax.ShapeDtypeStruct((NC, NPAD, H), jnp.float32),
    mesh=_MESH,
    scratch_types=[
        pltpu.VMEM((CH,), jnp.int32),
        pltpu.VMEM((CH,), jnp.int32),
        pltpu.VMEM((CH, H), jnp.float32),
        pltpu.VMEM((CH, H), jnp.float32),
        pltpu.SemaphoreType.DMA,
        pltpu.VMEM_SHARED((NPAD, H), jnp.float32),
    ],
    compiler_params=_SC_PARAMS,
)
def _msg_kernel(src_hbm, dst_hbm, feat_hbm, out_hbm, srcv, dstv, rows, zbuf,
                sem, acc):
    cid = lax.axis_index("c")
    sid = lax.axis_index("s")
    wid = cid * NS + sid

    @pl.loop(0, CH)
    def _zrow(i):
        @pl.loop(0, H // 16)
        def _zcol(j):
            zbuf[i, pl.ds(j * 16, 16)] = jnp.zeros((16,), jnp.float32)

    for i in range(ROWS_PER_TILE // CH):
        pltpu.sync_copy(zbuf, acc.at[pl.ds(sid * ROWS_PER_TILE + i * CH, CH)])
    plsc.subcore_barrier()

    base = wid * EPT

    @pl.loop(0, MSG_STEPS)
    def _step(j):
        off = base + j * CH
        pltpu.sync_copy(src_hbm.at[pl.ds(off, CH)], srcv)
        pltpu.sync_copy(dst_hbm.at[pl.ds(off, CH)], dstv)
        pltpu.async_copy(feat_hbm.at[srcv], rows, sem).wait()
        pltpu.sync_copy(rows, acc.at[dstv], add=True)

    plsc.subcore_barrier()
    for i in range(ROWS_PER_TILE // CH):
        r0 = sid * ROWS_PER_TILE + i * CH
        pltpu.sync_copy(acc.at[pl.ds(r0, CH)], out_hbm.at[cid, pl.ds(r0, CH)])


_PREC = lax.Precision.HIGHEST


def _gelu(x):
    return 0.5 * x * (1.0 + lax.erf(x * 0.7071067811865476))


def _tc_a_body(x_ref, w0_ref, rout_ref, y_ref):
    y0 = jnp.dot(x_ref[...], w0_ref[...], precision=_PREC)
    y_ref[...] = y0 * rout_ref[...]


def _tc_b_body(p_ref, rin_ref, rout_ref, b0_ref, w1_ref, z_ref):
    m = p_ref[0, :N] + p_ref[1, :N]
    h = _gelu(m * rin_ref[...] + b0_ref[...])
    z_ref[...] = jnp.dot(h, w1_ref[...], precision=_PREC) * rout_ref[...]


def _tc_c_body(p_ref, rin_ref, b1_ref, wl_ref, bl_ref, o_ref):
    m = p_ref[0, :N] + p_ref[1, :N]
    h = _gelu(m * rin_ref[...] + b1_ref[...])
    o_ref[...] = jnp.dot(h, wl_ref[...], precision=_PREC) + bl_ref[...]


_tc_a = pl.pallas_call(
    _tc_a_body,
    out_shape=jax.ShapeDtypeStruct((N, H), jnp.float32),
)

_tc_b = pl.pallas_call(
    _tc_b_body,
    out_shape=jax.ShapeDtypeStruct((N, H), jnp.float32),
)

_tc_c = pl.pallas_call(
    _tc_c_body,
    out_shape=jax.ShapeDtypeStruct((N, C), jnp.float32),
)


def kernel(x, edge_index, batch_num_nodes, W0, b0, W1, b1, Wlin, blin):
    src = edge_index[0]
    dst = edge_index[1]

    deg_idx = jnp.concatenate([
        src, dst + NPAD,
        jnp.full((DEG_EPAD - 2 * E,), DEG_DUMMY, jnp.int32),
    ])
    degp = _deg_kernel(deg_idx)
    deg = degp[0] + degp[1]
    rout = lax.rsqrt(jnp.maximum(deg[:N], 1.0)).reshape(N, 1)
    rin = lax.rsqrt(jnp.maximum(deg[NPAD:NPAD + N], 1.0)).reshape(N, 1)

    src_p = jnp.concatenate([src, jnp.zeros((EPAD - E,), jnp.int32)])
    dst_p = jnp.concatenate([dst, jnp.full((EPAD - E,), MSG_DUMMY_DST,
                                           jnp.int32)])

    y0s = _tc_a(x, W0, rout)
    p1 = _msg_kernel(src_p, dst_p, y0s)
    z = _tc_b(p1, rin, rout, b0.reshape(1, H), W1)
    p2 = _msg_kernel(src_p, dst_p, z)
    out_full = _tc_c(p2, rin, b1.reshape(1, H), Wlin, blin.reshape(1, C))

    offsets = jnp.concatenate([
        jnp.zeros((1,), jnp.int32),
        jnp.cumsum(batch_num_nodes)[:-1].astype(jnp.int32),
    ])
    return out_full[offsets]


# speedup vs baseline: 4.4027x; 1.0893x over previous
import functools

import jax
import jax.numpy as jnp
from jax import lax
from jax.experimental import pallas as pl
from jax.experimental.pallas import tpu as pltpu
from jax.experimental.pallas import tpu_sc as plsc

N = 10000
E = 320000
D = 128
H = 64
C = 40
G = 10

NC, NS = 2, 16
NW = NC * NS
CH = 128

NPAD = 10240
ROWS_PER_TILE = NPAD // NS

DEG_LEN = 2 * NPAD
DEG_STEPS = 158
DEG_PER_TILE = DEG_STEPS * CH
DEG_EPAD = DEG_PER_TILE * NW
DEG_SLICE = DEG_LEN // NS
DEG_DUMMY = 10100

NBUF = 4
MSG_STEPS = 80
EPT = MSG_STEPS * CH
EPAD = EPT * NW
STEADY = MSG_STEPS - NBUF
MSG_DUMMY_DST = 10200

_MESH = plsc.VectorSubcoreMesh(core_axis_name="c", subcore_axis_name="s")
_SC_PARAMS = pltpu.CompilerParams(use_tc_tiling_on_sc=False)


@functools.partial(
    pl.kernel,
    out_type=jax.ShapeDtypeStruct((NC, DEG_LEN), jnp.float32),
    mesh=_MESH,
    scratch_types=[
        pltpu.VMEM((DEG_STEPS, CH), jnp.int32),
        pltpu.VMEM((CH,), jnp.float32),
        pltpu.VMEM((DEG_SLICE,), jnp.float32),
        pltpu.VMEM_SHARED((DEG_LEN,), jnp.float32),
    ],
    compiler_params=_SC_PARAMS,
)
def _deg_kernel(idx_hbm, out_hbm, idxs, ones_v, zer_v, acc):
    cid = lax.axis_index("c")
    sid = lax.axis_index("s")
    wid = cid * NS + sid

    @pl.loop(0, DEG_SLICE // 16)
    def _zero(i):
        zer_v[pl.ds(i * 16, 16)] = jnp.zeros((16,), jnp.float32)

    @pl.loop(0, CH // 16)
    def _ones(i):
        ones_v[pl.ds(i * 16, 16)] = jnp.ones((16,), jnp.float32)

    pltpu.sync_copy(zer_v, acc.at[pl.ds(sid * DEG_SLICE, DEG_SLICE)])
    pltpu.sync_copy(idx_hbm.at[wid], idxs)
    plsc.subcore_barrier()

    @pl.loop(0, DEG_STEPS)
    def _step(j):
        pltpu.sync_copy(ones_v, acc.at[idxs.at[j]], add=True)

    plsc.subcore_barrier()
    pltpu.sync_copy(acc.at[pl.ds(sid * DEG_SLICE, DEG_SLICE)],
                    out_hbm.at[cid, pl.ds(sid * DEG_SLICE, DEG_SLICE)])


@functools.partial(
    pl.kernel,
    out_type=jax.ShapeDtypeStruct((NC, NPAD, H), jnp.float32),
    mesh=_MESH,
    scratch_types=[
        pltpu.VMEM((MSG_STEPS, CH), jnp.int32),
        pltpu.VMEM((MSG_STEPS, CH), jnp.int32),
        pltpu.VMEM((CH, H), jnp.float32),
        pltpu.VMEM((CH, H), jnp.float32),
        pltpu.VMEM((CH, H), jnp.float32),
        pltpu.VMEM((CH, H), jnp.float32),
        pltpu.VMEM((CH, H), jnp.float32),
        pltpu.SemaphoreType.DMA,
        pltpu.SemaphoreType.DMA,
        pltpu.SemaphoreType.DMA,
        pltpu.SemaphoreType.DMA,
        pltpu.VMEM_SHARED((NPAD, H), jnp.float32),
    ],
    compiler_params=_SC_PARAMS,
)
def _msg_kernel(src_hbm, dst_hbm, feat_hbm, out_hbm, srcs, dsts,
                rows0, rows1, rows2, rows3, zbuf, sem0, sem1, sem2, sem3,
                acc):
    cid = lax.axis_index("c")
    sid = lax.axis_index("s")
    wid = cid * NS + sid
    bufs = [(rows0, sem0), (rows1, sem1), (rows2, sem2), (rows3, sem3)]

    @pl.loop(0, CH)
    def _zrow(i):
        @pl.loop(0, H // 16)
        def _zcol(j):
            zbuf[i, pl.ds(j * 16, 16)] = jnp.zeros((16,), jnp.float32)

    for i in range(ROWS_PER_TILE // CH):
        pltpu.sync_copy(zbuf, acc.at[pl.ds(sid * ROWS_PER_TILE + i * CH, CH)])
    plsc.subcore_barrier()

    pltpu.sync_copy(src_hbm.at[wid], srcs)
    pltpu.sync_copy(dst_hbm.at[wid], dsts)

    for b in range(NBUF):
        rows, sem = bufs[b]
        pltpu.async_copy(feat_hbm.at[srcs.at[b]], rows, sem)

    @pl.loop(0, STEADY, step=NBUF)
    def _step(j):
        for b in range(NBUF):
            rows, sem = bufs[b]
            pltpu.make_async_copy(feat_hbm.at[srcs.at[j + b]], rows,
                                  sem).wait()
            pltpu.sync_copy(rows, acc.at[dsts.at[j + b]], add=True)
            pltpu.async_copy(feat_hbm.at[srcs.at[j + b + NBUF]], rows, sem)

    for b in range(NBUF):
        rows, sem = bufs[b]
        c = STEADY + b
        pltpu.make_async_copy(feat_hbm.at[srcs.at[c]], rows, sem).wait()
        pltpu.sync_copy(rows, acc.at[dsts.at[c]], add=True)

    plsc.subcore_barrier()
    for i in range(ROWS_PER_TILE // CH):
        r0 = sid * ROWS_PER_TILE + i * CH
        pltpu.sync_copy(acc.at[pl.ds(r0, CH)], out_hbm.at[cid, pl.ds(r0, CH)])


_PREC = lax.Precision.HIGHEST


def _gelu(x):
    return 0.5 * x * (1.0 + lax.erf(x * 0.7071067811865476))


def _tc_a_body(x_ref, w0_ref, rout_ref, y_ref):
    y0 = jnp.dot(x_ref[...], w0_ref[...], precision=_PREC)
    y_ref[...] = y0 * rout_ref[...]


def _tc_b_body(p_ref, rin_ref, rout_ref, b0_ref, w1_ref, z_ref):
    m = p_ref[0, :N] + p_ref[1, :N]
    h = _gelu(m * rin_ref[...] + b0_ref[...])
    z_ref[...] = jnp.dot(h, w1_ref[...], precision=_PREC) * rout_ref[...]


def _tc_c_body(p_ref, rin_ref, b1_ref, wl_ref, bl_ref, o_ref):
    m = p_ref[0, :N] + p_ref[1, :N]
    h = _gelu(m * rin_ref[...] + b1_ref[...])
    o_ref[...] = jnp.dot(h, wl_ref[...], precision=_PREC) + bl_ref[...]


_tc_a = pl.pallas_call(
    _tc_a_body,
    out_shape=jax.ShapeDtypeStruct((N, H), jnp.float32),
)

_tc_b = pl.pallas_call(
    _tc_b_body,
    out_shape=jax.ShapeDtypeStruct((N, H), jnp.float32),
)

_tc_c = pl.pallas_call(
    _tc_c_body,
    out_shape=jax.ShapeDtypeStruct((N, C), jnp.float32),
)


def kernel(x, edge_index, batch_num_nodes, W0, b0, W1, b1, Wlin, blin):
    src = edge_index[0]
    dst = edge_index[1]

    deg_idx = jnp.concatenate([
        src, dst + NPAD,
        jnp.full((DEG_EPAD - 2 * E,), DEG_DUMMY, jnp.int32),
    ]).reshape(NW, DEG_STEPS, CH)
    degp = _deg_kernel(deg_idx)
    deg = degp[0] + degp[1]
    rout = lax.rsqrt(jnp.maximum(deg[:N], 1.0)).reshape(N, 1)
    rin = lax.rsqrt(jnp.maximum(deg[NPAD:NPAD + N], 1.0)).reshape(N, 1)

    src_p = jnp.concatenate([src, jnp.zeros((EPAD - E,), jnp.int32)])
    src_p = src_p.reshape(NW, MSG_STEPS, CH)
    dst_p = jnp.concatenate([dst, jnp.full((EPAD - E,), MSG_DUMMY_DST,
                                           jnp.int32)])
    dst_p = dst_p.reshape(NW, MSG_STEPS, CH)

    y0s = _tc_a(x, W0, rout)
    p1 = _msg_kernel(src_p, dst_p, y0s)
    z = _tc_b(p1, rin, rout, b0.reshape(1, H), W1)
    p2 = _msg_kernel(src_p, dst_p, z)
    out_full = _tc_c(p2, rin, b1.reshape(1, H), Wlin, blin.reshape(1, C))

    offsets = jnp.concatenate([
        jnp.zeros((1,), jnp.int32),
        jnp.cumsum(batch_num_nodes)[:-1].astype(jnp.int32),
    ])
    return out_full[offsets]


# speedup vs baseline: 13.5527x; 3.0783x over previous
import functools

import jax
import jax.numpy as jnp
from jax import lax
from jax.experimental import pallas as pl
from jax.experimental.pallas import tpu as pltpu
from jax.experimental.pallas import tpu_sc as plsc

N = 10000
E = 320000
D = 128
H = 64
C = 40
G = 10

NC, NS = 2, 16
NW = NC * NS
CH = 128

NPAD = 10240
ROWS_PER_TILE = NPAD // NS

DEG_LEN = 2 * NPAD
DEG_STEPS = 158
DEG_PER_TILE = DEG_STEPS * CH
DEG_EPAD = DEG_PER_TILE * NW
DEG_SLICE = DEG_LEN // NS

NBUF = 4
MSG_STEPS = 80
EPT = MSG_STEPS * CH
EPAD = EPT * NW
STEADY = MSG_STEPS - NBUF

_PAD_SPREAD = NPAD - N

_MESH = plsc.VectorSubcoreMesh(core_axis_name="c", subcore_axis_name="s")
_SC_PARAMS = pltpu.CompilerParams(use_tc_tiling_on_sc=False)


@functools.partial(
    pl.kernel,
    out_type=jax.ShapeDtypeStruct((NC, DEG_LEN), jnp.float32),
    mesh=_MESH,
    scratch_types=[
        pltpu.VMEM((DEG_STEPS, CH), jnp.int32),
        pltpu.VMEM((CH,), jnp.float32),
        pltpu.VMEM((DEG_SLICE,), jnp.float32),
        pltpu.VMEM_SHARED((DEG_LEN,), jnp.float32),
    ],
    compiler_params=_SC_PARAMS,
)
def _deg_kernel(idx_hbm, out_hbm, idxs, ones_v, zer_v, acc):
    cid = lax.axis_index("c")
    sid = lax.axis_index("s")
    wid = cid * NS + sid

    @pl.loop(0, DEG_SLICE // 16)
    def _zero(i):
        zer_v[pl.ds(i * 16, 16)] = jnp.zeros((16,), jnp.float32)

    @pl.loop(0, CH // 16)
    def _ones(i):
        ones_v[pl.ds(i * 16, 16)] = jnp.ones((16,), jnp.float32)

    pltpu.sync_copy(zer_v, acc.at[pl.ds(sid * DEG_SLICE, DEG_SLICE)])
    pltpu.sync_copy(idx_hbm.at[wid], idxs)
    plsc.subcore_barrier()

    @pl.loop(0, DEG_STEPS)
    def _step(j):
        pltpu.sync_copy(ones_v, acc.at[idxs.at[j]], add=True)

    plsc.subcore_barrier()
    pltpu.sync_copy(acc.at[pl.ds(sid * DEG_SLICE, DEG_SLICE)],
                    out_hbm.at[cid, pl.ds(sid * DEG_SLICE, DEG_SLICE)])


@functools.partial(
    pl.kernel,
    out_type=jax.ShapeDtypeStruct((NC, NPAD, H), jnp.float32),
    mesh=_MESH,
    scratch_types=[
        pltpu.VMEM((MSG_STEPS, CH), jnp.int32),
        pltpu.VMEM((MSG_STEPS, CH), jnp.int32),
        pltpu.VMEM((CH, H), jnp.float32),
        pltpu.VMEM((CH, H), jnp.float32),
        pltpu.VMEM((CH, H), jnp.float32),
        pltpu.VMEM((CH, H), jnp.float32),
        pltpu.VMEM((CH, H), jnp.float32),
        pltpu.SemaphoreType.DMA,
        pltpu.SemaphoreType.DMA,
        pltpu.SemaphoreType.DMA,
        pltpu.SemaphoreType.DMA,
        pltpu.VMEM_SHARED((NPAD, H), jnp.float32),
    ],
    compiler_params=_SC_PARAMS,
)
def _msg_kernel(src_hbm, dst_hbm, feat_hbm, out_hbm, srcs, dsts,
                rows0, rows1, rows2, rows3, zbuf, sem0, sem1, sem2, sem3,
                acc):
    cid = lax.axis_index("c")
    sid = lax.axis_index("s")
    wid = cid * NS + sid
    bufs = [(rows0, sem0), (rows1, sem1), (rows2, sem2), (rows3, sem3)]

    @pl.loop(0, CH)
    def _zrow(i):
        @pl.loop(0, H // 16)
        def _zcol(j):
            zbuf[i, pl.ds(j * 16, 16)] = jnp.zeros((16,), jnp.float32)

    for i in range(ROWS_PER_TILE // CH):
        pltpu.sync_copy(zbuf, acc.at[pl.ds(sid * ROWS_PER_TILE + i * CH, CH)])
    plsc.subcore_barrier()

    pltpu.sync_copy(src_hbm.at[wid], srcs)
    pltpu.sync_copy(dst_hbm.at[wid], dsts)

    for b in range(NBUF):
        rows, sem = bufs[b]
        pltpu.async_copy(feat_hbm.at[srcs.at[b]], rows, sem)

    @pl.loop(0, STEADY, step=NBUF)
    def _step(j):
        for b in range(NBUF):
            rows, sem = bufs[b]
            pltpu.make_async_copy(feat_hbm.at[srcs.at[j + b]], rows,
                                  sem).wait()
            pltpu.sync_copy(rows, acc.at[dsts.at[j + b]], add=True)
            pltpu.async_copy(feat_hbm.at[srcs.at[j + b + NBUF]], rows, sem)

    for b in range(NBUF):
        rows, sem = bufs[b]
        c = STEADY + b
        pltpu.make_async_copy(feat_hbm.at[srcs.at[c]], rows, sem).wait()
        pltpu.sync_copy(rows, acc.at[dsts.at[c]], add=True)

    plsc.subcore_barrier()
    for i in range(ROWS_PER_TILE // CH):
        r0 = sid * ROWS_PER_TILE + i * CH
        pltpu.sync_copy(acc.at[pl.ds(r0, CH)], out_hbm.at[cid, pl.ds(r0, CH)])


_PREC = lax.Precision.HIGHEST


def _gelu(x):
    return 0.5 * x * (1.0 + lax.erf(x * 0.7071067811865476))


def _tc_a_body(x_ref, w0_ref, rout_ref, y_ref):
    y0 = jnp.dot(x_ref[...], w0_ref[...], precision=_PREC)
    y_ref[...] = y0 * rout_ref[...]


def _tc_b_body(p_ref, rin_ref, rout_ref, b0_ref, w1_ref, z_ref):
    m = p_ref[0, :N] + p_ref[1, :N]
    h = _gelu(m * rin_ref[...] + b0_ref[...])
    z_ref[...] = jnp.dot(h, w1_ref[...], precision=_PREC) * rout_ref[...]


def _tc_c_body(p_ref, rin_ref, b1_ref, wl_ref, bl_ref, o_ref):
    m = p_ref[0, :N] + p_ref[1, :N]
    h = _gelu(m * rin_ref[...] + b1_ref[...])
    o_ref[...] = jnp.dot(h, wl_ref[...], precision=_PREC) + bl_ref[...]


_tc_a = pl.pallas_call(
    _tc_a_body,
    out_shape=jax.ShapeDtypeStruct((N, H), jnp.float32),
)

_tc_b = pl.pallas_call(
    _tc_b_body,
    out_shape=jax.ShapeDtypeStruct((N, H), jnp.float32),
)

_tc_c = pl.pallas_call(
    _tc_c_body,
    out_shape=jax.ShapeDtypeStruct((N, C), jnp.float32),
)


def kernel(x, edge_index, batch_num_nodes, W0, b0, W1, b1, Wlin, blin):
    src = edge_index[0]
    dst = edge_index[1]

    deg_pad = N + (jnp.arange(DEG_EPAD - 2 * E, dtype=jnp.int32)
                   % _PAD_SPREAD)
    deg_idx = jnp.concatenate([src, dst + NPAD, deg_pad])
    deg_idx = deg_idx.reshape(NW, DEG_STEPS, CH)
    degp = _deg_kernel(deg_idx)
    deg = degp[0] + degp[1]
    rout = lax.rsqrt(jnp.maximum(deg[:N], 1.0)).reshape(N, 1)
    rin = lax.rsqrt(jnp.maximum(deg[NPAD:NPAD + N], 1.0)).reshape(N, 1)

    epad_ar = jnp.arange(EPAD - E, dtype=jnp.int32)
    src_p = jnp.concatenate([src, epad_ar % CH])
    src_p = src_p.reshape(NW, MSG_STEPS, CH)
    dst_p = jnp.concatenate([dst, N + epad_ar % _PAD_SPREAD])
    dst_p = dst_p.reshape(NW, MSG_STEPS, CH)

    y0s = _tc_a(x, W0, rout)
    p1 = _msg_kernel(src_p, dst_p, y0s)
    z = _tc_b(p1, rin, rout, b0.reshape(1, H), W1)
    p2 = _msg_kernel(src_p, dst_p, z)
    out_full = _tc_c(p2, rin, b1.reshape(1, H), Wlin, blin.reshape(1, C))

    offsets = jnp.concatenate([
        jnp.zeros((1,), jnp.int32),
        jnp.cumsum(batch_num_nodes)[:-1].astype(jnp.int32),
    ])
    return out_full[offsets]
